# baseline (speedup 1.0000x reference)
import functools

import jax
import jax.numpy as jnp
from jax.experimental import pallas as pl
from jax.experimental.pallas import tpu as pltpu

TM = 1024


def _gcn_body(a_ref, h_ref, w_ref, bias_ref, o_ref, *, d):
    a = a_ref[0]
    ah = jnp.dot(a, h_ref[0], preferred_element_type=jnp.float32)
    ah2 = ah.reshape(-1, d)
    out = jax.lax.dot_general(
        ah2, w_ref[...], (((1,), (1,)), ((), ())),
        preferred_element_type=jnp.float32)
    out = jnp.maximum(out + bias_ref[...], 0.0)
    o_ref[0] = out.reshape(a.shape[0], -1)


def kernel(prop_state, A, W, b):
    B, N, L, D = prop_state.shape
    H = prop_state.reshape(B, N, L * D)
    bias = b.reshape(1, D)

    grid = (B, N // TM)
    out = pl.pallas_call(
        functools.partial(_gcn_body, d=D),
        grid=grid,
        in_specs=[
            pl.BlockSpec((1, TM, N), lambda bi, i: (bi, i, 0)),
            pl.BlockSpec((1, N, L * D), lambda bi, i: (bi, 0, 0)),
            pl.BlockSpec((D, D), lambda bi, i: (0, 0)),
            pl.BlockSpec((1, D), lambda bi, i: (0, 0)),
        ],
        out_specs=pl.BlockSpec((1, TM, L * D), lambda bi, i: (bi, i, 0)),
        out_shape=jax.ShapeDtypeStruct((B, N, L * D), jnp.float32),
        compiler_params=pltpu.CompilerParams(
            dimension_semantics=("parallel", "parallel")),
    )(A, H, W, bias)
    return out.reshape(B, N, L, D)


# speedup vs baseline: 1.0033x; 1.0033x over previous
import functools

import jax
import jax.numpy as jnp
from jax.experimental import pallas as pl
from jax.experimental.pallas import tpu as pltpu

TM = 1024


def _gcn_body(a_ref, h_ref, w_ref, bias_ref, o_ref, *, d):
    a = a_ref[0]
    ah = jnp.dot(a, h_ref[pl.program_id(0)], preferred_element_type=jnp.float32)
    ah2 = ah.reshape(-1, d)
    out = jax.lax.dot_general(
        ah2, w_ref[...], (((1,), (1,)), ((), ())),
        preferred_element_type=jnp.float32)
    out = jnp.maximum(out + bias_ref[...], 0.0)
    o_ref[0] = out.reshape(a.shape[0], -1)


def kernel(prop_state, A, W, b):
    B, N, L, D = prop_state.shape
    H = prop_state.reshape(B, N, L * D)
    bias = b.reshape(1, D)

    grid = (B, N // TM)
    out = pl.pallas_call(
        functools.partial(_gcn_body, d=D),
        grid=grid,
        in_specs=[
            pl.BlockSpec((1, TM, N), lambda bi, i: (bi, i, 0)),
            pl.BlockSpec((4, N, L * D), lambda bi, i: (0, 0, 0)),
            pl.BlockSpec((D, D), lambda bi, i: (0, 0)),
            pl.BlockSpec((1, D), lambda bi, i: (0, 0)),
        ],
        out_specs=pl.BlockSpec((1, TM, L * D), lambda bi, i: (bi, i, 0)),
        out_shape=jax.ShapeDtypeStruct((B, N, L * D), jnp.float32),
        compiler_params=pltpu.CompilerParams(
            dimension_semantics=("parallel", "parallel")),
    )(A, H, W, bias)
    return out.reshape(B, N, L, D)
